# baseline (speedup 1.0000x reference)
import functools

import jax
import jax.numpy as jnp
import numpy as np
from jax import lax
from jax.experimental import pallas as pl
from jax.experimental.pallas import tpu as pltpu
from jax.experimental.pallas import tpu_sc as plsc

N = 10000
E = 320000
D = 128
DE = 16
EPS = 1e-5

NUM_TILES = 16
CHUNK = 128
GRP = 4
CH_PER_TILE = 160
NGRP = CH_PER_TILE // GRP
EPT = CH_PER_TILE * CHUNK
EPAD = EPT * NUM_TILES
NCHC = EPAD // CHUNK
DUMP = N
BLK = 80
NPAD = 10112
ROWS_PT = NPAD // NUM_TILES
NBLK = N // BLK
RS = float(1.0 / np.sqrt(1.0 + EPS))


def _sc_g_body(x_hbm, gs_hbm, zg_hbm, g_out,
               idxb, rows0, rows1, acc_g, sem0, sem1):
    c = lax.axis_index("c")
    t = lax.axis_index("s")
    r0 = t * ROWS_PT
    pltpu.sync_copy(zg_hbm.at[pl.ds(0, CHUNK)], rows0)
    for off in range(0, ROWS_PT, CHUNK):
        sz = min(CHUNK, ROWS_PT - off)
        pltpu.sync_copy(rows0.at[pl.ds(0, sz)], acc_g.at[pl.ds(r0 + off, sz)])
    plsc.subcore_barrier()

    ch0 = c * NCHC + t * CH_PER_TILE
    rows = (rows0, rows1)
    sems = (sem0, sem1)

    def wait_gather(b):
        pltpu.make_async_copy(x_hbm.at[idxb.at[0]], rows[b], sems[b]).wait()

    def group(g, carry):
        pltpu.sync_copy(gs_hbm.at[pl.ds(2 * (ch0 + g * GRP), 2 * GRP)], idxb)
        pltpu.async_copy(x_hbm.at[idxb.at[0]], rows0, sem0)
        for j in range(GRP):
            b = j % 2
            if j + 1 < GRP:
                pltpu.async_copy(x_hbm.at[idxb.at[2 * (j + 1)]],
                                 rows[1 - b], sems[1 - b])
            wait_gather(b)
            pltpu.sync_copy(rows[b], acc_g.at[idxb.at[2 * j + 1]], add=True)
        return carry

    lax.fori_loop(0, NGRP, group, 0)
    plsc.subcore_barrier()

    for off in range(0, ROWS_PT, CHUNK):
        sz = min(CHUNK, ROWS_PT - off)
        pltpu.sync_copy(acc_g.at[pl.ds(r0 + off, sz)], rows0.at[pl.ds(0, sz)])
        pltpu.sync_copy(rows0.at[pl.ds(0, sz)], g_out.at[c, pl.ds(r0 + off, sz)])


_sc_g = pl.kernel(
    _sc_g_body,
    out_type=jax.ShapeDtypeStruct((2, NPAD, D), jnp.float32),
    mesh=plsc.VectorSubcoreMesh(core_axis_name="c", subcore_axis_name="s"),
    scratch_types=[
        pltpu.VMEM((2 * GRP, CHUNK), jnp.int32),
        pltpu.VMEM((CHUNK, D), jnp.float32),
        pltpu.VMEM((CHUNK, D), jnp.float32),
        pltpu.VMEM_SHARED((NPAD, D), jnp.float32),
        pltpu.SemaphoreType.DMA,
        pltpu.SemaphoreType.DMA,
    ],
)


def _sc_a_body(ea_hbm, gs_hbm, zg_hbm, a_out,
               idxb, ea0, ea1, pad_v, acc_a, sem0, sem1):
    c = lax.axis_index("c")
    t = lax.axis_index("s")
    r0 = t * ROWS_PT
    pltpu.sync_copy(zg_hbm.at[pl.ds(0, CHUNK)], pad_v)
    for off in range(0, ROWS_PT, CHUNK):
        sz = min(CHUNK, ROWS_PT - off)
        pltpu.sync_copy(pad_v.at[pl.ds(0, sz)], acc_a.at[pl.ds(r0 + off, sz)])
    plsc.subcore_barrier()

    ch0 = c * NCHC + t * CH_PER_TILE
    eab = (ea0, ea1)
    sems = (sem0, sem1)

    def ea_base(k):
        return jnp.minimum(t * EPT + k * CHUNK, E - CHUNK)

    def start_ea(k, b):
        pltpu.async_copy(ea_hbm.at[pl.ds(ea_base(k), CHUNK)], eab[b], sems[b])

    def wait_ea(b):
        pltpu.make_async_copy(ea_hbm.at[pl.ds(0, CHUNK)], eab[b], sems[b]).wait()

    def group(g, carry):
        pltpu.sync_copy(gs_hbm.at[pl.ds(2 * (ch0 + g * GRP), 2 * GRP)], idxb)
        start_ea(g * GRP, 0)
        for j in range(GRP):
            b = j % 2
            if j + 1 < GRP:
                start_ea(g * GRP + j + 1, 1 - b)
            wait_ea(b)
            for r in range(CHUNK):
                pad_v[r, pl.ds(0, DE)] = eab[b][r, :]
            pltpu.sync_copy(pad_v, acc_a.at[idxb.at[2 * j + 1]], add=True)
        return carry

    lax.fori_loop(0, NGRP, group, 0)
    plsc.subcore_barrier()

    for off in range(0, ROWS_PT, CHUNK):
        sz = min(CHUNK, ROWS_PT - off)
        pltpu.sync_copy(acc_a.at[pl.ds(r0 + off, sz)], pad_v.at[pl.ds(0, sz)])
        pltpu.sync_copy(pad_v.at[pl.ds(0, sz)], a_out.at[c, pl.ds(r0 + off, sz)])


_sc_a = pl.kernel(
    _sc_a_body,
    out_type=jax.ShapeDtypeStruct((2, NPAD, D), jnp.float32),
    mesh=plsc.VectorSubcoreMesh(core_axis_name="c", subcore_axis_name="s"),
    scratch_types=[
        pltpu.VMEM((2 * GRP, CHUNK), jnp.int32),
        pltpu.VMEM((CHUNK, DE), jnp.float32),
        pltpu.VMEM((CHUNK, DE), jnp.float32),
        pltpu.VMEM((CHUNK, D), jnp.float32),
        pltpu.VMEM_SHARED((NPAD, D), jnp.float32),
        pltpu.SemaphoreType.DMA,
        pltpu.SemaphoreType.DMA,
    ],
)


def _dense_body(gi, ai, gj, aj, xr, wn, wne, wns, bnr, we, wee, wes, ber,
                gam, bet, hi, hj):
    hp = jax.lax.Precision.HIGHEST
    xb = xr[...]
    scale = gam[...] * RS
    shift = bet[...]
    pre_i = (jnp.dot(gi[0], wn[...], precision=hp, preferred_element_type=jnp.float32)
             + jnp.dot(ai[0], wne[...], precision=hp, preferred_element_type=jnp.float32)
             + jnp.dot(xb, wns[...], precision=hp, preferred_element_type=jnp.float32)
             + bnr[...])
    hi[...] = jnp.maximum(pre_i, 0.0) * scale + shift
    pre_j = (jnp.dot(gj[0], we[...], precision=hp, preferred_element_type=jnp.float32)
             + jnp.dot(aj[0], wee[...], precision=hp, preferred_element_type=jnp.float32)
             + jnp.dot(xb, wes[...], precision=hp, preferred_element_type=jnp.float32)
             + ber[...])
    hj[...] = jnp.maximum(pre_j, 0.0) * scale + shift


def _row_spec(shape, off):
    return pl.BlockSpec(shape, lambda i, o=off: (i + o, 0))


def _full_spec(shape):
    return pl.BlockSpec(shape, lambda i: (0, 0))


_dense = pl.pallas_call(
    _dense_body,
    grid=(NBLK,),
    in_specs=[
        pl.BlockSpec((1, BLK, D), lambda i: (0, i, 0)),
        pl.BlockSpec((1, BLK, D), lambda i: (0, i, 0)),
        pl.BlockSpec((1, BLK, D), lambda i: (1, i, 0)),
        pl.BlockSpec((1, BLK, D), lambda i: (1, i, 0)),
        _row_spec((BLK, D), 0),
        _full_spec((D, D)),
        _full_spec((D, D)),
        _full_spec((D, D)),
        _full_spec((1, D)),
        _full_spec((D, D)),
        _full_spec((D, D)),
        _full_spec((D, D)),
        _full_spec((1, D)),
        _full_spec((1, D)),
        _full_spec((1, D)),
    ],
    out_specs=[
        pl.BlockSpec((BLK, D), lambda i: (i, 0)),
        pl.BlockSpec((BLK, D), lambda i: (i, 0)),
    ],
    out_shape=[
        jax.ShapeDtypeStruct((N, D), jnp.float32),
        jax.ShapeDtypeStruct((N, D), jnp.float32),
    ],
)


def kernel(x, edge_index, edge_attr, Wn, Wn_edge, Wn_self, bn, We, We_edge,
           We_self, be, gamma, beta):
    src = edge_index[0]
    dst = edge_index[1]
    padg = jnp.zeros((EPAD - E,), jnp.int32)
    pads = jnp.full((EPAD - E,), DUMP, jnp.int32)
    gcat = jnp.concatenate([dst, padg, src, padg])
    scat = jnp.concatenate([src, pads, dst, pads])
    gs = jnp.stack([gcat.reshape(-1, CHUNK), scat.reshape(-1, CHUNK)],
                   axis=1).reshape(-1, CHUNK)
    zg = jnp.zeros((CHUNK, D), jnp.float32)
    g_out = _sc_g(x, gs, zg)
    a_out = _sc_a(edge_attr, gs, zg)
    wne = jnp.pad(Wn_edge, ((0, D - DE), (0, 0)))
    wee = jnp.pad(We_edge, ((0, D - DE), (0, 0)))
    hi, hj = _dense(g_out, a_out, g_out, a_out, x,
                    Wn, wne, Wn_self, bn.reshape(1, D),
                    We, wee, We_self, be.reshape(1, D),
                    gamma.reshape(1, D), beta.reshape(1, D))
    return hi, hj


# speedup vs baseline: 1.1691x; 1.1691x over previous
import functools

import jax
import jax.numpy as jnp
import numpy as np
from jax import lax
from jax.experimental import pallas as pl
from jax.experimental.pallas import tpu as pltpu
from jax.experimental.pallas import tpu_sc as plsc

N = 10000
E = 320000
D = 128
DE = 16
EPS = 1e-5

NUM_TILES = 16
CHUNK = 128
CH_PER_TILE = 157
EPT = CH_PER_TILE * CHUNK
EPAD = EPT * NUM_TILES
DUMP = N
BLK = 80
NPAD = 10240
ROWS_PT = NPAD // NUM_TILES
NBLK = N // BLK
RS = float(1.0 / np.sqrt(1.0 + EPS))


def _sc_g_body(x_hbm, gcat_hbm, scat_hbm, zg_hbm, g_out,
               idxg_v, idxs_v, rows_v, acc_g, sem):
    c = lax.axis_index("c")
    t = lax.axis_index("s")
    r0 = t * ROWS_PT
    pltpu.sync_copy(zg_hbm.at[pl.ds(0, CHUNK)], rows_v)
    for off in range(0, ROWS_PT, CHUNK):
        pltpu.sync_copy(rows_v, acc_g.at[pl.ds(r0 + off, CHUNK)])
    plsc.subcore_barrier()

    tile_base = c * EPAD + t * EPT

    def chunk_body(k, carry):
        base = tile_base + k * CHUNK
        pltpu.sync_copy(gcat_hbm.at[pl.ds(base, CHUNK)], idxg_v)
        pltpu.sync_copy(scat_hbm.at[pl.ds(base, CHUNK)], idxs_v)
        pltpu.async_copy(x_hbm.at[idxg_v], rows_v, sem).wait()
        pltpu.sync_copy(rows_v, acc_g.at[idxs_v], add=True)
        return carry

    lax.fori_loop(0, CH_PER_TILE, chunk_body, 0)
    plsc.subcore_barrier()

    o0 = c * NPAD + r0
    for off in range(0, ROWS_PT, CHUNK):
        pltpu.sync_copy(acc_g.at[pl.ds(r0 + off, CHUNK)], rows_v)
        pltpu.sync_copy(rows_v, g_out.at[pl.ds(o0 + off, CHUNK)])


_sc_g = pl.kernel(
    _sc_g_body,
    out_type=jax.ShapeDtypeStruct((2 * NPAD, D), jnp.float32),
    mesh=plsc.VectorSubcoreMesh(core_axis_name="c", subcore_axis_name="s"),
    scratch_types=[
        pltpu.VMEM((CHUNK,), jnp.int32),
        pltpu.VMEM((CHUNK,), jnp.int32),
        pltpu.VMEM((CHUNK, D), jnp.float32),
        pltpu.VMEM_SHARED((NPAD, D), jnp.float32),
        pltpu.SemaphoreType.DMA,
    ],
)


def _sc_a_body(ea_hbm, scat_hbm, zg_hbm, a_out, idxs_v, ea_v, pad_v, acc_a):
    c = lax.axis_index("c")
    t = lax.axis_index("s")
    r0 = t * ROWS_PT
    pltpu.sync_copy(zg_hbm.at[pl.ds(0, CHUNK)], pad_v)
    for off in range(0, ROWS_PT, CHUNK):
        pltpu.sync_copy(pad_v, acc_a.at[pl.ds(r0 + off, CHUNK)])
    plsc.subcore_barrier()

    tile_base = c * EPAD + t * EPT

    def chunk_body(k, carry):
        base = tile_base + k * CHUNK
        ebase = jnp.minimum(t * EPT + k * CHUNK, E - CHUNK)
        pltpu.sync_copy(scat_hbm.at[pl.ds(base, CHUNK)], idxs_v)
        pltpu.sync_copy(ea_hbm.at[pl.ds(ebase, CHUNK)], ea_v)
        for r in range(CHUNK):
            pad_v[r, pl.ds(0, DE)] = ea_v[r, :]
        pltpu.sync_copy(pad_v, acc_a.at[idxs_v], add=True)
        return carry

    lax.fori_loop(0, CH_PER_TILE, chunk_body, 0)
    plsc.subcore_barrier()

    o0 = c * NPAD + r0
    for off in range(0, ROWS_PT, CHUNK):
        pltpu.sync_copy(acc_a.at[pl.ds(r0 + off, CHUNK)], pad_v)
        pltpu.sync_copy(pad_v, a_out.at[pl.ds(o0 + off, CHUNK)])


_sc_a = pl.kernel(
    _sc_a_body,
    out_type=jax.ShapeDtypeStruct((2 * NPAD, D), jnp.float32),
    mesh=plsc.VectorSubcoreMesh(core_axis_name="c", subcore_axis_name="s"),
    scratch_types=[
        pltpu.VMEM((CHUNK,), jnp.int32),
        pltpu.VMEM((CHUNK, DE), jnp.float32),
        pltpu.VMEM((CHUNK, D), jnp.float32),
        pltpu.VMEM_SHARED((NPAD, D), jnp.float32),
    ],
)


def _dense_body(gi, ai, gj, aj, xr, wn, wne, wns, bnr, we, wee, wes, ber,
                gam, bet, hi, hj):
    hp = jax.lax.Precision.HIGHEST
    xb = xr[...]
    scale = gam[...] * RS
    shift = bet[...]
    pre_i = (jnp.dot(gi[...], wn[...], precision=hp, preferred_element_type=jnp.float32)
             + jnp.dot(ai[...], wne[...], precision=hp, preferred_element_type=jnp.float32)
             + jnp.dot(xb, wns[...], precision=hp, preferred_element_type=jnp.float32)
             + bnr[...])
    hi[...] = jnp.maximum(pre_i, 0.0) * scale + shift
    pre_j = (jnp.dot(gj[...], we[...], precision=hp, preferred_element_type=jnp.float32)
             + jnp.dot(aj[...], wee[...], precision=hp, preferred_element_type=jnp.float32)
             + jnp.dot(xb, wes[...], precision=hp, preferred_element_type=jnp.float32)
             + ber[...])
    hj[...] = jnp.maximum(pre_j, 0.0) * scale + shift


def _row_spec(shape, off):
    return pl.BlockSpec(shape, lambda i, o=off: (i + o, 0))


def _full_spec(shape):
    return pl.BlockSpec(shape, lambda i: (0, 0))


_dense = pl.pallas_call(
    _dense_body,
    grid=(NBLK,),
    in_specs=[
        _row_spec((BLK, D), 0),
        _row_spec((BLK, D), 0),
        _row_spec((BLK, D), NPAD // BLK),
        _row_spec((BLK, D), NPAD // BLK),
        _row_spec((BLK, D), 0),
        _full_spec((D, D)),
        _full_spec((D, D)),
        _full_spec((D, D)),
        _full_spec((1, D)),
        _full_spec((D, D)),
        _full_spec((D, D)),
        _full_spec((D, D)),
        _full_spec((1, D)),
        _full_spec((1, D)),
        _full_spec((1, D)),
    ],
    out_specs=[
        pl.BlockSpec((BLK, D), lambda i: (i, 0)),
        pl.BlockSpec((BLK, D), lambda i: (i, 0)),
    ],
    out_shape=[
        jax.ShapeDtypeStruct((N, D), jnp.float32),
        jax.ShapeDtypeStruct((N, D), jnp.float32),
    ],
)


def kernel(x, edge_index, edge_attr, Wn, Wn_edge, Wn_self, bn, We, We_edge,
           We_self, be, gamma, beta):
    src = edge_index[0]
    dst = edge_index[1]
    padg = jnp.zeros((EPAD - E,), jnp.int32)
    pads = jnp.full((EPAD - E,), DUMP, jnp.int32)
    gcat = jnp.concatenate([dst, padg, src, padg])
    scat = jnp.concatenate([src, pads, dst, pads])
    zg = jnp.zeros((CHUNK, D), jnp.float32)
    g_out = _sc_g(x, gcat, scat, zg)
    a_out = _sc_a(edge_attr, scat, zg)
    wne = jnp.pad(Wn_edge, ((0, D - DE), (0, 0)))
    wee = jnp.pad(We_edge, ((0, D - DE), (0, 0)))
    hi, hj = _dense(g_out, a_out, g_out, a_out, x,
                    Wn, wne, Wn_self, bn.reshape(1, D),
                    We, wee, We_self, be.reshape(1, D),
                    gamma.reshape(1, D), beta.reshape(1, D))
    return hi, hj
